# Initial kernel scaffold; baseline (speedup 1.0000x reference)
import jax
import jax.numpy as jnp
from jax.experimental import pallas as pl


def kernel(cos_theta, cos_theta_m, target, x_norm):
    raise NotImplementedError("write your pallas kernel here")


# speedup vs baseline: 1.5473x; 1.5473x over previous
import functools

import jax
import jax.numpy as jnp
from jax import lax
from jax.experimental import pallas as pl
from jax.experimental.pallas import tpu as pltpu

U_A = 110.0


def _mag_body(num_blocks, bv, v, tgt_ref, vm_ref, xn_ref, ct_ref,
              out_ref, loss_ref, lossg_ref, m_ref, s_ref):
    j = pl.program_id(0)
    b = ct_ref.shape[0]

    @pl.when(j == 0)
    def _init():
        m_ref[...] = jnp.full_like(m_ref, -jnp.inf)
        s_ref[...] = jnp.zeros_like(s_ref)

    c = ct_ref[...]
    cols = j * bv + lax.broadcasted_iota(jnp.int32, (b, bv), 1)
    t = tgt_ref[...]
    blk = jnp.where(cols == t, vm_ref[...], c)
    out_ref[...] = blk

    valid = cols < v
    mblk = jnp.where(valid, blk, -jnp.inf)
    bm = jnp.max(mblk, axis=1, keepdims=True)
    m_old = m_ref[...]
    m_new = jnp.maximum(m_old, bm)
    e = jnp.exp(mblk - m_new)
    s_new = s_ref[...] * jnp.exp(m_old - m_new) + jnp.sum(e, axis=1,
                                                          keepdims=True)
    m_ref[...] = m_new
    s_ref[...] = s_new

    @pl.when(j == num_blocks - 1)
    def _finish():
        log_z = m_new + jnp.log(s_new)
        picked = vm_ref[...]
        loss_ref[...] = (jnp.sum(log_z - picked) / b).reshape(1, 1)
        xn = xn_ref[...]
        lossg_ref[...] = (jnp.sum(xn * (1.0 / (U_A * U_A)) + 1.0 / xn)
                          / b).reshape(1, 1)


def _mag_loss_tc(cos_theta, target, vals_m, x_norm, bv=2048):
    b, v = cos_theta.shape
    num_blocks = pl.cdiv(v, bv)
    grid = (num_blocks,)
    kernel_fn = functools.partial(_mag_body, num_blocks, bv, v)
    out, loss, loss_g = pl.pallas_call(
        kernel_fn,
        grid=grid,
        in_specs=[
            pl.BlockSpec((b, 1), lambda j: (0, 0)),
            pl.BlockSpec((b, 1), lambda j: (0, 0)),
            pl.BlockSpec((b, 1), lambda j: (0, 0)),
            pl.BlockSpec((b, bv), lambda j: (0, j)),
        ],
        out_specs=[
            pl.BlockSpec((b, bv), lambda j: (0, j)),
            pl.BlockSpec((1, 1), lambda j: (0, 0)),
            pl.BlockSpec((1, 1), lambda j: (0, 0)),
        ],
        out_shape=[
            jax.ShapeDtypeStruct((b, v), jnp.float32),
            jax.ShapeDtypeStruct((1, 1), jnp.float32),
            jax.ShapeDtypeStruct((1, 1), jnp.float32),
        ],
        scratch_shapes=[
            pltpu.VMEM((b, 1), jnp.float32),
            pltpu.VMEM((b, 1), jnp.float32),
        ],
    )(target[:, None], vals_m[:, None], x_norm[:, None], cos_theta)
    return out, loss[0, 0], loss_g[0, 0]


def kernel(cos_theta, cos_theta_m, target, x_norm):
    b, v = cos_theta.shape
    vals_m = jnp.take_along_axis(cos_theta_m, target[:, None], axis=1)[:, 0]
    out, loss, loss_g = _mag_loss_tc(cos_theta, target, vals_m, x_norm)
    return (loss, loss_g, out)
